# Initial kernel scaffold; baseline (speedup 1.0000x reference)
import functools

import jax
import jax.numpy as jnp
from jax import lax
from jax.experimental import pallas as pl
from jax.experimental.pallas import tpu as pltpu
from jax.experimental.pallas import tpu_sc as plsc

N = 10000
E = 320000
D = 128
H = 128
C = 64

NC = 2
NS = 16
NW = NC * NS
CH = 128
NCHUNK = 79
EPT = CH * NCHUNK
E_PAD = EPT * NW
ROWS_PT = N // NS

_MESH = plsc.VectorSubcoreMesh(core_axis_name="c", subcore_axis_name="s")


@functools.partial(
    pl.kernel,
    mesh=_MESH,
    out_type=jax.ShapeDtypeStruct((NC, N, 16), jnp.float32),
    scratch_types=[
        pltpu.VMEM((CH,), jnp.int32),
        pltpu.VMEM((CH,), jnp.float32),
        pltpu.VMEM((CH, 16), jnp.float32),
        pltpu.VMEM_SHARED((N, 16), jnp.float32),
    ],
)
def _deg_kernel(dst_hbm, w_hbm, out_hbm, dstv, wv, wtile, accum):
    cid = lax.axis_index("c")
    sid = lax.axis_index("s")
    wid = cid * NS + sid

    def zero_row(i, carry):
        wtile[i, :] = jnp.zeros((16,), jnp.float32)
        return carry

    lax.fori_loop(0, CH, zero_row, 0)
    r0 = sid * ROWS_PT
    for k in range(5):
        pltpu.sync_copy(wtile.at[pl.ds(0, 125)],
                        accum.at[pl.ds(r0 + k * 125, 125)])
    plsc.subcore_barrier()

    base = wid * EPT

    def chunk(ci, carry):
        off = base + ci * CH
        pltpu.sync_copy(dst_hbm.at[pl.ds(off, CH)], dstv)
        pltpu.sync_copy(w_hbm.at[pl.ds(off, CH)], wv)

        def fill(e, c2):
            wvec = plsc.load_gather(wv, [jnp.full((16,), e, jnp.int32)])
            wtile[e, :] = wvec
            return c2

        lax.fori_loop(0, CH, fill, 0)
        pltpu.sync_copy(wtile, accum.at[dstv], add=True)
        return carry

    lax.fori_loop(0, NCHUNK, chunk, 0)
    plsc.subcore_barrier()
    pltpu.sync_copy(accum.at[pl.ds(r0, ROWS_PT)],
                    out_hbm.at[cid, pl.ds(r0, ROWS_PT)])


def _make_edge_kernel(F):
    nreg = F // 16

    @functools.partial(
        pl.kernel,
        mesh=_MESH,
        out_type=jax.ShapeDtypeStruct((NC, N, F), jnp.float32),
        scratch_types=[
            pltpu.VMEM((CH,), jnp.int32),
            pltpu.VMEM((CH,), jnp.int32),
            pltpu.VMEM((CH,), jnp.float32),
            pltpu.VMEM((CH, F), jnp.float32),
            pltpu.VMEM_SHARED((N, F), jnp.float32),
            pltpu.SemaphoreType.DMA,
        ],
    )
    def edge_kernel(hin, src_hbm, dst_hbm, w_hbm, out_hbm,
                    srcv, dstv, wv, rows, accum, sem):
        cid = lax.axis_index("c")
        sid = lax.axis_index("s")
        wid = cid * NS + sid

        def zero_row(i, carry):
            for j in range(nreg):
                rows[i, pl.ds(16 * j, 16)] = jnp.zeros((16,), jnp.float32)
            return carry

        lax.fori_loop(0, 125, zero_row, 0)
        r0 = sid * ROWS_PT
        for k in range(5):
            pltpu.sync_copy(rows.at[pl.ds(0, 125)],
                            accum.at[pl.ds(r0 + k * 125, 125)])
        plsc.subcore_barrier()

        base = wid * EPT

        def chunk(ci, carry):
            off = base + ci * CH
            pltpu.sync_copy(src_hbm.at[pl.ds(off, CH)], srcv)
            pltpu.sync_copy(dst_hbm.at[pl.ds(off, CH)], dstv)
            pltpu.sync_copy(w_hbm.at[pl.ds(off, CH)], wv)
            pltpu.async_copy(hin.at[srcv], rows, sem).wait()

            def scale(e, c2):
                wvec = plsc.load_gather(wv, [jnp.full((16,), e, jnp.int32)])
                for j in range(nreg):
                    sl = pl.ds(16 * j, 16)
                    rows[e, sl] = rows[e, sl] * wvec
                return c2

            lax.fori_loop(0, CH, scale, 0)
            pltpu.sync_copy(rows, accum.at[dstv], add=True)
            return carry

        lax.fori_loop(0, NCHUNK, chunk, 0)
        plsc.subcore_barrier()
        pltpu.sync_copy(accum.at[pl.ds(r0, ROWS_PT)],
                        out_hbm.at[cid, pl.ds(r0, ROWS_PT)])

    return edge_kernel


_edge128 = _make_edge_kernel(H)
_edge64 = _make_edge_kernel(C)


BR = 1000


def _tc1_body(x_ref, w1t_ref, degp_ref, hs_ref, dinv_ref):
    deg = degp_ref[0, :, 0:1] + degp_ref[1, :, 0:1]
    dinv = jnp.where(deg > 0, lax.rsqrt(jnp.maximum(deg, 1e-12)), 0.0)
    h = jnp.dot(x_ref[...], w1t_ref[...], preferred_element_type=jnp.float32)
    hs_ref[...] = h * dinv
    dinv_ref[...] = dinv


def _tc1(x, w1t, degp):
    return pl.pallas_call(
        _tc1_body,
        grid=(N // BR,),
        in_specs=[
            pl.BlockSpec((BR, D), lambda i: (i, 0)),
            pl.BlockSpec((D, H), lambda i: (0, 0)),
            pl.BlockSpec((NC, BR, 16), lambda i: (0, i, 0)),
        ],
        out_specs=[
            pl.BlockSpec((BR, H), lambda i: (i, 0)),
            pl.BlockSpec((BR, 1), lambda i: (i, 0)),
        ],
        out_shape=[
            jax.ShapeDtypeStruct((N, H), jnp.float32),
            jax.ShapeDtypeStruct((N, 1), jnp.float32),
        ],
    )(x, w1t, degp)


def _tc2_body(p_ref, dinv_ref, b1_ref, w2t_ref, gs_ref):
    a = (p_ref[0] + p_ref[1]) * dinv_ref[...] + b1_ref[...]
    a = jnp.maximum(a, 0.0)
    g = jnp.dot(a, w2t_ref[...], preferred_element_type=jnp.float32)
    gs_ref[...] = g * dinv_ref[...]


def _tc2(p, dinv, b1, w2t):
    return pl.pallas_call(
        _tc2_body,
        grid=(N // BR,),
        in_specs=[
            pl.BlockSpec((NC, BR, H), lambda i: (0, i, 0)),
            pl.BlockSpec((BR, 1), lambda i: (i, 0)),
            pl.BlockSpec((1, H), lambda i: (0, 0)),
            pl.BlockSpec((H, C), lambda i: (0, 0)),
        ],
        out_specs=pl.BlockSpec((BR, C), lambda i: (i, 0)),
        out_shape=jax.ShapeDtypeStruct((N, C), jnp.float32),
    )(p, dinv, b1, w2t)


def _tc3_body(q_ref, dinv_ref, b2_ref, logits_ref, soft_ref):
    lg = (q_ref[0] + q_ref[1]) * dinv_ref[...] + b2_ref[...]
    logits_ref[...] = lg
    m = jnp.max(lg, axis=1, keepdims=True)
    ex = jnp.exp(lg - m)
    soft_ref[...] = ex / jnp.sum(ex, axis=1, keepdims=True)


def _tc3(q, dinv, b2):
    return pl.pallas_call(
        _tc3_body,
        grid=(N // BR,),
        in_specs=[
            pl.BlockSpec((NC, BR, C), lambda i: (0, i, 0)),
            pl.BlockSpec((BR, 1), lambda i: (i, 0)),
            pl.BlockSpec((1, C), lambda i: (0, 0)),
        ],
        out_specs=[
            pl.BlockSpec((BR, C), lambda i: (i, 0)),
            pl.BlockSpec((BR, C), lambda i: (i, 0)),
        ],
        out_shape=[
            jax.ShapeDtypeStruct((N, C), jnp.float32),
            jax.ShapeDtypeStruct((N, C), jnp.float32),
        ],
    )(q, dinv, b2)


@jax.jit
def kernel(x, edge_index, edge_weight, W1, b1, W2, b2):
    src = edge_index[0]
    dst = edge_index[1]
    npad = E_PAD - E
    pad_idx = (jnp.arange(npad, dtype=jnp.int32) * 37) % N
    srcp = jnp.concatenate([src, pad_idx])
    dstp = jnp.concatenate([dst, pad_idx])
    wp = jnp.concatenate([edge_weight, jnp.zeros((npad,), jnp.float32)])

    degp = _deg_kernel(dstp, wp)
    hs, dinv = _tc1(x, W1.T, degp)
    p = _edge128(hs, srcp, dstp, wp)
    gs = _tc2(p, dinv, b1.reshape(1, H), W2.T)
    q = _edge64(gs, srcp, dstp, wp)
    logits, soft = _tc3(q, dinv, b2.reshape(1, C))
    return (logits, soft)


# speedup vs baseline: 8.2176x; 8.2176x over previous
import functools

import jax
import jax.numpy as jnp
from jax import lax
from jax.experimental import pallas as pl
from jax.experimental.pallas import tpu as pltpu
from jax.experimental.pallas import tpu_sc as plsc

N = 10000
E = 320000
D = 128
H = 128
C = 64

NC = 2
NS = 16
NW = NC * NS
CH = 128
NCHUNK = 79
EPT = CH * NCHUNK
E_PAD = EPT * NW
RPT = 632
N_PAD = RPT * NS

_MESH = plsc.VectorSubcoreMesh(core_axis_name="c", subcore_axis_name="s",
                               num_cores=NC, num_subcores=NS)


@functools.partial(
    pl.kernel,
    mesh=_MESH,
    compiler_params=pltpu.CompilerParams(needs_layout_passes=False),
    out_type=jax.ShapeDtypeStruct((NC, N_PAD, 128), jnp.float32),
    scratch_types=[
        pltpu.VMEM((CH,), jnp.int32),
        pltpu.VMEM((CH,), jnp.float32),
        pltpu.VMEM((CH, 128), jnp.float32),
        pltpu.VMEM_SHARED((N_PAD, 128), jnp.float32),
    ],
)
def _deg_kernel(dst_hbm, w_hbm, out_hbm, dstv, wv, wtile, accum):
    cid = lax.axis_index("c")
    sid = lax.axis_index("s")
    wid = cid * NS + sid

    def zero_row(i, carry):
        for j in range(8):
            wtile[i, pl.ds(16 * j, 16)] = jnp.zeros((16,), jnp.float32)
        return carry

    lax.fori_loop(0, CH, zero_row, 0)
    r0 = sid * RPT
    for k in range(4):
        pltpu.sync_copy(wtile, accum.at[pl.ds(r0 + k * CH, CH)])
    pltpu.sync_copy(wtile.at[pl.ds(0, 120)],
                    accum.at[pl.ds(r0 + 4 * CH, 120)])
    plsc.subcore_barrier()

    base = wid * EPT

    def chunk(ci, carry):
        off = base + ci * CH
        pltpu.sync_copy(dst_hbm.at[pl.ds(off, CH)], dstv)
        pltpu.sync_copy(w_hbm.at[pl.ds(off, CH)], wv)

        def fill(e, c2):
            wtile[e, pl.ds(0, 16)] = plsc.load_gather(
                wv, [jnp.full((16,), e, jnp.int32)])
            return c2

        lax.fori_loop(0, CH, fill, 0)
        pltpu.sync_copy(wtile, accum.at[dstv], add=True)
        return carry

    lax.fori_loop(0, NCHUNK, chunk, 0)
    plsc.subcore_barrier()
    pltpu.sync_copy(accum.at[pl.ds(r0, RPT)],
                    out_hbm.at[cid, pl.ds(r0, RPT)])


def _make_edge_kernel(F):
    nreg = F // 16

    @functools.partial(
        pl.kernel,
        mesh=_MESH,
        compiler_params=pltpu.CompilerParams(needs_layout_passes=False),
        out_type=jax.ShapeDtypeStruct((NC, N_PAD, F), jnp.float32),
        scratch_types=[
            pltpu.VMEM((CH,), jnp.int32),
            pltpu.VMEM((CH,), jnp.int32),
            pltpu.VMEM((CH,), jnp.float32),
            pltpu.VMEM((CH, F), jnp.float32),
            pltpu.VMEM_SHARED((N_PAD, F), jnp.float32),
            pltpu.SemaphoreType.DMA,
        ],
    )
    def edge_kernel(hin, src_hbm, dst_hbm, w_hbm, out_hbm,
                    srcv, dstv, wv, rows, accum, sem):
        cid = lax.axis_index("c")
        sid = lax.axis_index("s")
        wid = cid * NS + sid

        def zero_row(i, carry):
            for j in range(nreg):
                rows[i, pl.ds(16 * j, 16)] = jnp.zeros((16,), jnp.float32)
            return carry

        lax.fori_loop(0, CH, zero_row, 0)
        r0 = sid * RPT
        for k in range(4):
            pltpu.sync_copy(rows, accum.at[pl.ds(r0 + k * CH, CH)])
        pltpu.sync_copy(rows.at[pl.ds(0, 120)],
                        accum.at[pl.ds(r0 + 4 * CH, 120)])
        plsc.subcore_barrier()

        base = wid * EPT

        def chunk(ci, carry):
            off = base + ci * CH
            pltpu.sync_copy(src_hbm.at[pl.ds(off, CH)], srcv)
            pltpu.sync_copy(dst_hbm.at[pl.ds(off, CH)], dstv)
            pltpu.sync_copy(w_hbm.at[pl.ds(off, CH)], wv)
            pltpu.async_copy(hin.at[srcv], rows, sem).wait()

            def scale(e, c2):
                wvec = plsc.load_gather(wv, [jnp.full((16,), e, jnp.int32)])
                for j in range(nreg):
                    sl = pl.ds(16 * j, 16)
                    rows[e, sl] = rows[e, sl] * wvec
                return c2

            lax.fori_loop(0, CH, scale, 0)
            pltpu.sync_copy(rows, accum.at[dstv], add=True)
            return carry

        lax.fori_loop(0, NCHUNK, chunk, 0)
        plsc.subcore_barrier()
        pltpu.sync_copy(accum.at[pl.ds(r0, RPT)],
                        out_hbm.at[cid, pl.ds(r0, RPT)])

    return edge_kernel


_edge128 = _make_edge_kernel(H)


BR = 1000


def _tc1_body(x_ref, w1t_ref, degp_ref, hs_ref, dinv_ref):
    deg = degp_ref[0, :, 0:1] + degp_ref[1, :, 0:1]
    dinv = jnp.where(deg > 0, lax.rsqrt(jnp.maximum(deg, 1e-12)), 0.0)
    h = jnp.dot(x_ref[...], w1t_ref[...], preferred_element_type=jnp.float32)
    hs_ref[...] = h * dinv
    dinv_ref[...] = dinv


def _tc1(x, w1t, degp):
    return pl.pallas_call(
        _tc1_body,
        grid=(N // BR,),
        in_specs=[
            pl.BlockSpec((BR, D), lambda i: (i, 0)),
            pl.BlockSpec((D, H), lambda i: (0, 0)),
            pl.BlockSpec((NC, BR, 128), lambda i: (0, i, 0)),
        ],
        out_specs=[
            pl.BlockSpec((BR, H), lambda i: (i, 0)),
            pl.BlockSpec((BR, 1), lambda i: (i, 0)),
        ],
        out_shape=[
            jax.ShapeDtypeStruct((N, H), jnp.float32),
            jax.ShapeDtypeStruct((N, 1), jnp.float32),
        ],
    )(x, w1t, degp)


def _tc2_body(p_ref, dinv_ref, b1_ref, a2_ref):
    conv1 = (p_ref[0] + p_ref[1]) * dinv_ref[...] + b1_ref[...]
    a2_ref[...] = jnp.maximum(conv1, 0.0) * dinv_ref[...]


def _tc2(p, dinv, b1):
    return pl.pallas_call(
        _tc2_body,
        grid=(N // BR,),
        in_specs=[
            pl.BlockSpec((NC, BR, H), lambda i: (0, i, 0)),
            pl.BlockSpec((BR, 1), lambda i: (i, 0)),
            pl.BlockSpec((1, H), lambda i: (0, 0)),
        ],
        out_specs=pl.BlockSpec((BR, H), lambda i: (i, 0)),
        out_shape=jax.ShapeDtypeStruct((N, H), jnp.float32),
    )(p, dinv, b1)


def _tc3_body(q_ref, dinv_ref, b2_ref, w2t_ref, logits_ref, soft_ref):
    agg = (q_ref[0] + q_ref[1]) * dinv_ref[...]
    lg = jnp.dot(agg, w2t_ref[...],
                 preferred_element_type=jnp.float32) + b2_ref[...]
    logits_ref[...] = lg
    m = jnp.max(lg, axis=1, keepdims=True)
    ex = jnp.exp(lg - m)
    soft_ref[...] = ex / jnp.sum(ex, axis=1, keepdims=True)


def _tc3(q, dinv, b2, w2t):
    return pl.pallas_call(
        _tc3_body,
        grid=(N // BR,),
        in_specs=[
            pl.BlockSpec((NC, BR, H), lambda i: (0, i, 0)),
            pl.BlockSpec((BR, 1), lambda i: (i, 0)),
            pl.BlockSpec((1, C), lambda i: (0, 0)),
            pl.BlockSpec((H, C), lambda i: (0, 0)),
        ],
        out_specs=[
            pl.BlockSpec((BR, C), lambda i: (i, 0)),
            pl.BlockSpec((BR, C), lambda i: (i, 0)),
        ],
        out_shape=[
            jax.ShapeDtypeStruct((N, C), jnp.float32),
            jax.ShapeDtypeStruct((N, C), jnp.float32),
        ],
    )(q, dinv, b2, w2t)


@jax.jit
def kernel(x, edge_index, edge_weight, W1, b1, W2, b2):
    src = edge_index[0]
    dst = edge_index[1]
    npad = E_PAD - E
    pad_idx = (jnp.arange(npad, dtype=jnp.int32) * 37) % N
    srcp = jnp.concatenate([src, pad_idx])
    dstp = jnp.concatenate([dst, pad_idx])
    wp = jnp.concatenate([edge_weight, jnp.zeros((npad,), jnp.float32)])

    degp = _deg_kernel(dstp, wp)
    hs, dinv = _tc1(x, W1.T, degp)
    p = _edge128(hs, srcp, dstp, wp)
    a2 = _tc2(p, dinv, b1.reshape(1, H))
    q = _edge128(a2, srcp, dstp, wp)
    logits, soft = _tc3(q, dinv, b2.reshape(1, C), W2.T)
    return (logits, soft)
